# baseline (speedup 1.0000x reference)
import functools

import jax
import jax.numpy as jnp
from jax import lax
from jax.experimental import pallas as pl
from jax.experimental.pallas import tpu as pltpu
from jax.experimental.pallas import tpu_sc as plsc

_NC = 2
_NS = 16
_NW = _NC * _NS
_LANES = 16
_CHUNK = 64


def _row_log_softmax(x):
    m = jnp.max(x, axis=-1, keepdims=True)
    s = x - m
    lse = jnp.log(jnp.sum(jnp.exp(s), axis=-1, keepdims=True))
    return s - lse


def _proj_loss_body(cell_ref, gene_ref, peak_ref, gcs_ref, pcs_ref, w_ref,
                    b_ref, gp_ref, ppb_ref, loss_ref):
    h = gene_ref.shape[1]
    gene = gene_ref[...]
    peak = peak_ref[...]
    gp_ref[...] = lax.dot_general(gene, w_ref[:, :h], (((1,), (1,)), ((), ())),
                                  preferred_element_type=jnp.float32)
    ppb_ref[...] = lax.dot_general(peak, w_ref[:, h:], (((1,), (1,)), ((), ())),
                                   preferred_element_type=jnp.float32) + b_ref[...]
    cell = cell_ref[...]
    dec1 = lax.dot_general(gene, cell, (((1,), (1,)), ((), ())),
                           preferred_element_type=jnp.float32)
    dec2 = lax.dot_general(peak, cell, (((1,), (1,)), ((), ())),
                           preferred_element_type=jnp.float32)
    logp_x1 = _row_log_softmax(dec1)
    logp_x2 = _row_log_softmax(dec2)
    logp_y1 = _row_log_softmax(gcs_ref[...])
    logp_y2 = _row_log_softmax(pcs_ref[...])
    p_y1 = jnp.exp(logp_y1)
    p_y2 = jnp.exp(logp_y2)
    l1 = jnp.sum(p_y1 * (logp_y1 - logp_x1)) / (dec1.shape[0] * dec1.shape[1])
    l2 = jnp.sum(p_y2 * (logp_y2 - logp_x2)) / (dec2.shape[0] * dec2.shape[1])
    loss_ref[...] = jnp.reshape(l1 + l2, (1, 1))


def _expand_body(g, h, p_per_w, gp_hbm, ppb_hbm, out_hbm,
                 gp_v, pp_v, buf0, buf1, sem0, sem1):
    wid = lax.axis_index("s") * _NC + lax.axis_index("c")
    pltpu.sync_copy(gp_hbm, gp_v)
    pltpu.sync_copy(ppb_hbm.at[pl.ds(wid * p_per_w, p_per_w)], pp_v)
    bufs = (buf0, buf1)
    sems = (sem0, sem1)
    pending = [None, None]
    n_chunks = g // _CHUNK
    chunk = 0
    for pk in range(p_per_w):
        prow = [pp_v[pk, pl.ds(cg * _LANES, _LANES)]
                for cg in range(h // _LANES)]
        for half in range(n_chunks):
            slot = chunk % 2
            if pending[slot] is not None:
                pending[slot].wait()
            buf = bufs[slot]
            base = half * _CHUNK

            def _row(r, carry):
                for cg in range(h // _LANES):
                    v = gp_v[base + r, pl.ds(cg * _LANES, _LANES)] + prow[cg]
                    buf[r, pl.ds(cg * _LANES, _LANES)] = jnp.maximum(v, 0.0)
                return carry

            lax.fori_loop(0, _CHUNK, _row, 0, unroll=False)
            row0 = (wid * p_per_w + pk) * g + base
            cp = pltpu.make_async_copy(buf, out_hbm.at[pl.ds(row0, _CHUNK)],
                                       sems[slot])
            cp.start()
            pending[slot] = cp
            chunk += 1
    for slot in (0, 1):
        if pending[slot] is not None:
            pending[slot].wait()


def kernel(cell_emb, gene_emb, peak_emb, gene_cell_sub, peak_cell_sub, W, b):
    c, h = cell_emb.shape
    g = gene_emb.shape[0]
    p = peak_emb.shape[0]
    b2d = jnp.reshape(b, (1, h))

    gp, ppb, loss = pl.pallas_call(
        _proj_loss_body,
        out_shape=[
            jax.ShapeDtypeStruct((g, h), jnp.float32),
            jax.ShapeDtypeStruct((p, h), jnp.float32),
            jax.ShapeDtypeStruct((1, 1), jnp.float32),
        ],
    )(cell_emb, gene_emb, peak_emb, gene_cell_sub, peak_cell_sub, W, b2d)

    p_per_w = p // _NW
    mesh = plsc.VectorSubcoreMesh(core_axis_name="c", subcore_axis_name="s")
    expand = functools.partial(
        pl.kernel,
        mesh=mesh,
        out_type=jax.ShapeDtypeStruct((p * g, h), jnp.float32),
        scratch_types=[
            pltpu.VMEM((g, h), jnp.float32),
            pltpu.VMEM((p_per_w, h), jnp.float32),
            pltpu.VMEM((_CHUNK, h), jnp.float32),
            pltpu.VMEM((_CHUNK, h), jnp.float32),
            pltpu.SemaphoreType.DMA,
            pltpu.SemaphoreType.DMA,
        ],
    )(functools.partial(_expand_body, g, h, p_per_w))
    out = expand(gp, ppb)
    return out, jnp.reshape(loss, ())


# speedup vs baseline: 2.2015x; 2.2015x over previous
import functools

import jax
import jax.numpy as jnp
from jax import lax
from jax.experimental import pallas as pl
from jax.experimental.pallas import tpu as pltpu

_PB = 8


def _row_log_softmax(x):
    m = jnp.max(x, axis=-1, keepdims=True)
    s = x - m
    lse = jnp.log(jnp.sum(jnp.exp(s), axis=-1, keepdims=True))
    return s - lse


def _fused_body(cell_ref, gene_ref, peak_full_ref, gcs_ref, pcs_ref, w_ref,
                b_ref, peak_blk_ref, out_ref, loss_ref, gp_ref):
    i = pl.program_id(0)
    h = gene_ref.shape[1]
    g = gene_ref.shape[0]

    @pl.when(i == 0)
    def _():
        gene = gene_ref[...]
        gp_ref[...] = lax.dot_general(
            gene, w_ref[:, :h], (((1,), (1,)), ((), ())),
            preferred_element_type=jnp.float32)
        cell = cell_ref[...]
        dec1 = lax.dot_general(gene, cell, (((1,), (1,)), ((), ())),
                               preferred_element_type=jnp.float32)
        dec2 = lax.dot_general(peak_full_ref[...], cell,
                               (((1,), (1,)), ((), ())),
                               preferred_element_type=jnp.float32)
        logp_x1 = _row_log_softmax(dec1)
        logp_x2 = _row_log_softmax(dec2)
        logp_y1 = _row_log_softmax(gcs_ref[...])
        logp_y2 = _row_log_softmax(pcs_ref[...])
        p_y1 = jnp.exp(logp_y1)
        p_y2 = jnp.exp(logp_y2)
        l1 = jnp.sum(p_y1 * (logp_y1 - logp_x1)) / (dec1.shape[0] * dec1.shape[1])
        l2 = jnp.sum(p_y2 * (logp_y2 - logp_x2)) / (dec2.shape[0] * dec2.shape[1])
        loss_ref[...] = jnp.reshape(l1 + l2, (1, 1))

    pp = lax.dot_general(peak_blk_ref[...], w_ref[:, h:],
                         (((1,), (1,)), ((), ())),
                         preferred_element_type=jnp.float32) + b_ref[...]
    out = jnp.maximum(gp_ref[...][None, :, :] + pp[:, None, :], 0.0)
    out_ref[...] = jnp.reshape(out, (_PB * g, h))


def kernel(cell_emb, gene_emb, peak_emb, gene_cell_sub, peak_cell_sub, W, b):
    c, h = cell_emb.shape
    g = gene_emb.shape[0]
    p = peak_emb.shape[0]
    grid = p // _PB
    b2d = jnp.reshape(b, (1, h))

    full = lambda shape: pl.BlockSpec(shape, lambda i: (0, 0))
    out, loss = pl.pallas_call(
        _fused_body,
        grid=(grid,),
        in_specs=[
            full((c, h)),
            full((g, h)),
            full((p, h)),
            full((g, c)),
            full((p, c)),
            full((h, 2 * h)),
            full((1, h)),
            pl.BlockSpec((_PB, h), lambda i: (i, 0)),
        ],
        out_specs=[
            pl.BlockSpec((_PB * g, h), lambda i: (i, 0)),
            pl.BlockSpec((1, 1), lambda i: (0, 0)),
        ],
        out_shape=[
            jax.ShapeDtypeStruct((p * g, h), jnp.float32),
            jax.ShapeDtypeStruct((1, 1), jnp.float32),
        ],
        scratch_shapes=[pltpu.VMEM((g, h), jnp.float32)],
    )(cell_emb, gene_emb, peak_emb, gene_cell_sub, peak_cell_sub, W, b2d,
      peak_emb)
    return out, jnp.reshape(loss, ())


# speedup vs baseline: 2.7504x; 1.2493x over previous
import functools

import jax
import jax.numpy as jnp
from jax import lax
from jax.experimental import pallas as pl
from jax.experimental.pallas import tpu as pltpu

_PB = 16


def _row_log_softmax(x):
    m = jnp.max(x, axis=-1, keepdims=True)
    s = x - m
    lse = jnp.log(jnp.sum(jnp.exp(s), axis=-1, keepdims=True))
    return s - lse


def _fused_body(cell_ref, gene_ref, peak_full_ref, gcs_ref, pcs_ref, w_ref,
                b_ref, peak_blk_ref, out_ref, loss_ref, gp_ref):
    i = pl.program_id(0)
    h = gene_ref.shape[1]
    g = gene_ref.shape[0]

    @pl.when(i == 0)
    def _():
        gene = gene_ref[...]
        gp_ref[...] = lax.dot_general(
            gene, w_ref[:, :h], (((1,), (1,)), ((), ())),
            preferred_element_type=jnp.float32)
        cell = cell_ref[...]
        dec1 = lax.dot_general(gene, cell, (((1,), (1,)), ((), ())),
                               preferred_element_type=jnp.float32)
        dec2 = lax.dot_general(peak_full_ref[...], cell,
                               (((1,), (1,)), ((), ())),
                               preferred_element_type=jnp.float32)
        logp_x1 = _row_log_softmax(dec1)
        logp_x2 = _row_log_softmax(dec2)
        logp_y1 = _row_log_softmax(gcs_ref[...])
        logp_y2 = _row_log_softmax(pcs_ref[...])
        p_y1 = jnp.exp(logp_y1)
        p_y2 = jnp.exp(logp_y2)
        l1 = jnp.sum(p_y1 * (logp_y1 - logp_x1)) / (dec1.shape[0] * dec1.shape[1])
        l2 = jnp.sum(p_y2 * (logp_y2 - logp_x2)) / (dec2.shape[0] * dec2.shape[1])
        loss_ref[...] = jnp.reshape(l1 + l2, (1, 1))

    pp = lax.dot_general(peak_blk_ref[...], w_ref[:, h:],
                         (((1,), (1,)), ((), ())),
                         preferred_element_type=jnp.float32) + b_ref[...]
    out = jnp.maximum(gp_ref[...][None, :, :] + pp[:, None, :], 0.0)
    out_ref[...] = jnp.reshape(out, (_PB * g, h))


def kernel(cell_emb, gene_emb, peak_emb, gene_cell_sub, peak_cell_sub, W, b):
    c, h = cell_emb.shape
    g = gene_emb.shape[0]
    p = peak_emb.shape[0]
    grid = p // _PB
    b2d = jnp.reshape(b, (1, h))

    full = lambda shape: pl.BlockSpec(shape, lambda i: (0, 0))
    out, loss = pl.pallas_call(
        _fused_body,
        grid=(grid,),
        in_specs=[
            full((c, h)),
            full((g, h)),
            full((p, h)),
            full((g, c)),
            full((p, c)),
            full((h, 2 * h)),
            full((1, h)),
            pl.BlockSpec((_PB, h), lambda i: (i, 0)),
        ],
        out_specs=[
            pl.BlockSpec((_PB * g, h), lambda i: (i, 0)),
            pl.BlockSpec((1, 1), lambda i: (0, 0)),
        ],
        out_shape=[
            jax.ShapeDtypeStruct((p * g, h), jnp.float32),
            jax.ShapeDtypeStruct((1, 1), jnp.float32),
        ],
        scratch_shapes=[pltpu.VMEM((g, h), jnp.float32)],
    )(cell_emb, gene_emb, peak_emb, gene_cell_sub, peak_cell_sub, W, b2d,
      peak_emb)
    return out, jnp.reshape(loss, ())
